# baseline (speedup 1.0000x reference)
import functools

import jax
import jax.numpy as jnp
from jax import lax
from jax.experimental import pallas as pl
from jax.experimental.pallas import tpu as pltpu
from jax.experimental.pallas import tpu_sc as plsc

NUM_NODES = 100000
EMBED_DIM = 128
NUM_CORES = 2
CHUNK_ROWS = 2000
NUM_CHUNKS = NUM_NODES // CHUNK_ROWS
MAX_K = NUM_CHUNKS // NUM_CORES
NBUF = 2


def kernel(embedding_table):
    n, d = embedding_table.shape
    mesh = plsc.VectorSubcoreMesh(core_axis_name="c", subcore_axis_name="s")

    @functools.partial(
        pl.kernel,
        mesh=mesh,
        out_type=jax.ShapeDtypeStruct((n, d), embedding_table.dtype),
        scratch_types=[
            pltpu.VMEM_SHARED((NBUF, CHUNK_ROWS, EMBED_DIM), jnp.float32),
            pltpu.SemaphoreType.DMA((NBUF,)),
            pltpu.SemaphoreType.DMA((NBUF,)),
        ],
    )
    def copy_k(table_hbm, out_hbm, bufs, in_sems, out_sems):
        cid = lax.axis_index("c")
        sid = lax.axis_index("s")

        def in_dma(k, slot):
            c = cid + k * NUM_CORES
            return pltpu.make_async_copy(
                table_hbm.at[pl.ds(c * CHUNK_ROWS, CHUNK_ROWS)],
                bufs.at[slot],
                in_sems.at[slot],
            )

        def out_dma(k, slot):
            c = cid + k * NUM_CORES
            return pltpu.make_async_copy(
                bufs.at[slot],
                out_hbm.at[pl.ds(c * CHUNK_ROWS, CHUNK_ROWS)],
                out_sems.at[slot],
            )

        @pl.when(sid == 0)
        def _():
            for k in range(NBUF - 1):
                in_dma(k, k % NBUF).start()

            for k in range(MAX_K):
                slot = k % NBUF
                kp = k + NBUF - 1
                if kp < MAX_K:
                    prev = kp - NBUF
                    if prev >= 0:
                        out_dma(prev, kp % NBUF).wait()
                    in_dma(kp, kp % NBUF).start()
                in_dma(k, slot).wait()
                out_dma(k, slot).start()

            for k in range(max(0, MAX_K - NBUF), MAX_K):
                out_dma(k, k % NBUF).wait()

    return copy_k(embedding_table)


# speedup vs baseline: 1.2259x; 1.2259x over previous
import functools

import jax
import jax.numpy as jnp
from jax import lax
from jax.experimental import pallas as pl
from jax.experimental.pallas import tpu as pltpu
from jax.experimental.pallas import tpu_sc as plsc

NUM_NODES = 100000
EMBED_DIM = 128
NUM_CORES = 2

SP_ROWS = 44800
SP_CHUNK = 1600
SP_NCHUNK = SP_ROWS // SP_CHUNK
SP_MAXK = SP_NCHUNK // NUM_CORES
SP_NBUF = 2

ST_BASE = SP_ROWS
ST_ROWS = NUM_NODES - SP_ROWS
ST_CHUNK = 400
ST_NCHUNK = ST_ROWS // ST_CHUNK
ST_WORKERS = (16 - 1) * NUM_CORES
ST_MAXK = -(-ST_NCHUNK // ST_WORKERS)
ST_NBUF = 2


def kernel(embedding_table):
    n, d = embedding_table.shape
    mesh = plsc.VectorSubcoreMesh(core_axis_name="c", subcore_axis_name="s")

    @functools.partial(
        pl.kernel,
        mesh=mesh,
        out_type=jax.ShapeDtypeStruct((n, d), embedding_table.dtype),
        scratch_types=[
            pltpu.VMEM((ST_NBUF, ST_CHUNK, EMBED_DIM), jnp.float32),
            pltpu.SemaphoreType.DMA((ST_NBUF,)),
            pltpu.SemaphoreType.DMA((ST_NBUF,)),
            pltpu.VMEM_SHARED((SP_NBUF, SP_CHUNK, EMBED_DIM), jnp.float32),
            pltpu.SemaphoreType.DMA((SP_NBUF,)),
            pltpu.SemaphoreType.DMA((SP_NBUF,)),
        ],
    )
    def copy_k(table_hbm, out_hbm, st_bufs, st_isems, st_osems,
               sp_bufs, sp_isems, sp_osems):
        cid = lax.axis_index("c")
        sid = lax.axis_index("s")

        def sp_in(k, slot):
            c = cid + k * NUM_CORES
            return pltpu.make_async_copy(
                table_hbm.at[pl.ds(c * SP_CHUNK, SP_CHUNK)],
                sp_bufs.at[slot],
                sp_isems.at[slot],
            )

        def sp_out(k, slot):
            c = cid + k * NUM_CORES
            return pltpu.make_async_copy(
                sp_bufs.at[slot],
                out_hbm.at[pl.ds(c * SP_CHUNK, SP_CHUNK)],
                sp_osems.at[slot],
            )

        @pl.when(sid == 0)
        def _():
            for k in range(SP_NBUF - 1):
                sp_in(k, k % SP_NBUF).start()
            for k in range(SP_MAXK):
                slot = k % SP_NBUF
                kp = k + SP_NBUF - 1
                if kp < SP_MAXK:
                    prev = kp - SP_NBUF
                    if prev >= 0:
                        sp_out(prev, kp % SP_NBUF).wait()
                    sp_in(kp, kp % SP_NBUF).start()
                sp_in(k, slot).wait()
                sp_out(k, slot).start()
            for k in range(max(0, SP_MAXK - SP_NBUF), SP_MAXK):
                sp_out(k, k % SP_NBUF).wait()

        wid = (sid - 1) * NUM_CORES + cid

        def st_in(k, slot):
            c = wid + k * ST_WORKERS
            return pltpu.make_async_copy(
                table_hbm.at[pl.ds(ST_BASE + c * ST_CHUNK, ST_CHUNK)],
                st_bufs.at[slot],
                st_isems.at[slot],
            )

        def st_out(k, slot):
            c = wid + k * ST_WORKERS
            return pltpu.make_async_copy(
                st_bufs.at[slot],
                out_hbm.at[pl.ds(ST_BASE + c * ST_CHUNK, ST_CHUNK)],
                st_osems.at[slot],
            )

        def st_valid(k):
            return jnp.logical_and(sid >= 1, wid + k * ST_WORKERS < ST_NCHUNK)

        for k in range(min(ST_NBUF - 1, ST_MAXK)):
            @pl.when(st_valid(k))
            def _(k=k):
                st_in(k, k % ST_NBUF).start()

        for k in range(ST_MAXK):
            slot = k % ST_NBUF
            kp = k + ST_NBUF - 1
            if kp < ST_MAXK:
                @pl.when(st_valid(kp))
                def _(kp=kp):
                    prev = kp - ST_NBUF
                    if prev >= 0:
                        st_out(prev, kp % ST_NBUF).wait()
                    st_in(kp, kp % ST_NBUF).start()

            @pl.when(st_valid(k))
            def _(k=k, slot=slot):
                st_in(k, slot).wait()
                st_out(k, slot).start()

        for k in range(ST_MAXK):
            @pl.when(jnp.logical_and(st_valid(k),
                                     jnp.logical_not(st_valid(k + ST_NBUF))))
            def _(k=k):
                st_out(k, k % ST_NBUF).wait()

    return copy_k(embedding_table)


# speedup vs baseline: 1.2547x; 1.0235x over previous
import functools

import jax
import jax.numpy as jnp
from jax import lax
from jax.experimental import pallas as pl
from jax.experimental.pallas import tpu as pltpu
from jax.experimental.pallas import tpu_sc as plsc

NUM_NODES = 100000
EMBED_DIM = 128
NUM_CORES = 2
NUM_SUBCORES = 16
NUM_WORKERS = NUM_CORES * NUM_SUBCORES
CHUNK_ROWS = 400
NUM_CHUNKS = NUM_NODES // CHUNK_ROWS
MAX_K = -(-NUM_CHUNKS // NUM_WORKERS)
NBUF = 2


def kernel(embedding_table):
    n, d = embedding_table.shape
    mesh = plsc.VectorSubcoreMesh(core_axis_name="c", subcore_axis_name="s")

    @functools.partial(
        pl.kernel,
        mesh=mesh,
        out_type=jax.ShapeDtypeStruct((n, d), embedding_table.dtype),
        scratch_types=[
            pltpu.VMEM((NBUF, CHUNK_ROWS, EMBED_DIM), jnp.float32),
            pltpu.SemaphoreType.DMA((NBUF,)),
            pltpu.SemaphoreType.DMA((NBUF,)),
        ],
    )
    def copy_k(table_hbm, out_hbm, bufs, in_sems, out_sems):
        wid = lax.axis_index("s") * NUM_CORES + lax.axis_index("c")

        def in_dma(k, slot):
            c = wid + k * NUM_WORKERS
            return pltpu.make_async_copy(
                table_hbm.at[pl.ds(c * CHUNK_ROWS, CHUNK_ROWS)],
                bufs.at[slot],
                in_sems.at[slot],
            )

        def out_dma(k, slot):
            c = wid + k * NUM_WORKERS
            return pltpu.make_async_copy(
                bufs.at[slot],
                out_hbm.at[pl.ds(c * CHUNK_ROWS, CHUNK_ROWS)],
                out_sems.at[slot],
            )

        def valid(k):
            return wid + k * NUM_WORKERS < NUM_CHUNKS

        for k in range(min(NBUF - 1, MAX_K)):
            @pl.when(valid(k))
            def _(k=k):
                in_dma(k, k % NBUF).start()

        for k in range(MAX_K):
            slot = k % NBUF
            kp = k + NBUF - 1
            if kp < MAX_K:
                @pl.when(valid(kp))
                def _(kp=kp):
                    prev = kp - NBUF
                    if prev >= 0:
                        out_dma(prev, kp % NBUF).wait()
                    in_dma(kp, kp % NBUF).start()

            @pl.when(valid(k))
            def _(k=k, slot=slot):
                in_dma(k, slot).wait()
                out_dma(k, slot).start()

        for k in range(MAX_K):
            @pl.when(jnp.logical_and(valid(k), jnp.logical_not(valid(k + NBUF))))
            def _(k=k):
                out_dma(k, k % NBUF).wait()

    return copy_k(embedding_table)
